# baseline (speedup 1.0000x reference)
import functools

import jax
import jax.numpy as jnp
from jax import lax
from jax.experimental import pallas as pl
from jax.experimental.pallas import tpu as pltpu
from jax.experimental.pallas import tpu_sc as plsc

N = 10000
E = 640000
H = 64
G = 4
D4 = 4 * H
TH = 2 * H

NC = 2
NS = 16
EPS = E // NS
BLK = 80
NBLK = EPS // BLK
CH = 632
CH_LAST = N - (NS - 1) * CH

RA = 1000
F_IN = 1536


@functools.cache
def _edge_agg_kernel():
    mesh = plsc.VectorSubcoreMesh(
        core_axis_name="c", subcore_axis_name="s",
        num_cores=NC, num_subcores=NS)

    @functools.partial(
        pl.kernel,
        out_type=jax.ShapeDtypeStruct((NC, N, H), jnp.float32),
        mesh=mesh,
        scratch_types=[
            pltpu.VMEM((128, BLK), jnp.int32),
            pltpu.VMEM((128, BLK), jnp.int32),
            pltpu.VMEM((BLK, H), jnp.float32),
            pltpu.VMEM((BLK, H), jnp.float32),
            pltpu.VMEM((BLK, H), jnp.float32),
            pltpu.VMEM((BLK, H), jnp.float32),
            pltpu.VMEM_SHARED((N, H), jnp.float32),
            pltpu.VMEM_SHARED((N, H), jnp.float32),
            pltpu.SemaphoreType.DMA,
            pltpu.SemaphoreType.DMA,
            pltpu.SemaphoreType.DMA,
            pltpu.SemaphoreType.DMA,
        ],
        compiler_params=pltpu.CompilerParams(use_tc_tiling_on_sc=False),
    )
    def edge_agg(tabs, src, dst, zeros, out, sidx, didx,
                 rows0, rows1, rows2, rows3, tab_sp, acc,
                 sg0, sg1, sg2, sg3):
        c = lax.axis_index("c")
        s = lax.axis_index("s")
        row0 = pl.multiple_of(s * CH, 8)
        @pl.when(s < NS - 1)
        def _zero_main():
            pltpu.sync_copy(zeros, acc.at[pl.ds(row0, CH)])
            pltpu.sync_copy(tabs.at[c, pl.ds(row0, CH)],
                            tab_sp.at[pl.ds(row0, CH)])

        @pl.when(s == NS - 1)
        def _zero_last():
            pltpu.sync_copy(zeros.at[pl.ds(0, CH_LAST)],
                            acc.at[pl.ds((NS - 1) * CH, CH_LAST)])
            pltpu.sync_copy(tabs.at[c, pl.ds((NS - 1) * CH, CH_LAST)],
                            tab_sp.at[pl.ds((NS - 1) * CH, CH_LAST)])

        plsc.subcore_barrier()

        rows = (rows0, rows1, rows2, rows3)
        sg = (sg0, sg1, sg2, sg3)

        table = tabs.at[c]
        for off, nbh in ((0, 124), (124, 124), (248, 124), (372, 128)):
            pltpu.sync_copy(src.at[s, pl.ds(off, nbh)],
                            sidx.at[pl.ds(0, nbh)])
            pltpu.sync_copy(dst.at[s, pl.ds(off, nbh)],
                            didx.at[pl.ds(0, nbh)])
            for j in range(3):
                pltpu.async_copy(table.at[sidx.at[j]], rows[j], sg[j])

            def body(k, carry):
                for b in range(4):
                    i = 4 * k + b
                    pltpu.make_async_copy(table.at[sidx.at[i]], rows[b],
                                          sg[b]).wait()
                    pltpu.sync_copy(rows[b], acc.at[didx.at[i]], add=True)
                    bp = (b + 3) % 4

                    @pl.when(i + 3 < nbh)
                    def _prefetch():
                        if b == 0:
                            @pl.when(k % 2 == 1)
                            def _from_spmem():
                                pltpu.async_copy(tab_sp.at[sidx.at[i + 3]],
                                                 rows[bp], sg[bp])

                            @pl.when(k % 2 == 0)
                            def _from_hbm():
                                pltpu.async_copy(table.at[sidx.at[i + 3]],
                                                 rows[bp], sg[bp])
                        else:
                            pltpu.async_copy(table.at[sidx.at[i + 3]],
                                             rows[bp], sg[bp])
                return carry

            lax.fori_loop(0, nbh // 4, body, 0)

        plsc.subcore_barrier()

        @pl.when(s < NS - 1)
        def _write_main():
            pltpu.sync_copy(acc.at[pl.ds(row0, CH)],
                            out.at[c, pl.ds(row0, CH)])

        @pl.when(s == NS - 1)
        def _write_last():
            pltpu.sync_copy(acc.at[pl.ds((NS - 1) * CH, CH_LAST)],
                            out.at[c, pl.ds((NS - 1) * CH, CH_LAST)])

    return edge_agg


def _table(h, tvec):
    y = jnp.maximum(h, 0.0) + 1e-7
    u = jnp.exp(tvec * y)
    return jnp.stack([u, y * u], axis=0)


def _stage_a_body(x_ref, w_ref, b_ref, t_ref, h_ref, tab_ref):
    h = jnp.dot(x_ref[...], w_ref[...], preferred_element_type=jnp.float32)
    h = jnp.maximum(h + b_ref[...], 0.0)
    h_ref[...] = h
    tab_ref[...] = _table(h, t_ref[...])


@functools.cache
def _stage_a():
    return pl.pallas_call(
        _stage_a_body,
        grid=(N // RA,),
        in_specs=[
            pl.BlockSpec((RA, F_IN), lambda i: (i, 0)),
            pl.BlockSpec((F_IN, H), lambda i: (0, 0)),
            pl.BlockSpec((1, H), lambda i: (0, 0)),
            pl.BlockSpec((1, H), lambda i: (0, 0)),
        ],
        out_specs=[
            pl.BlockSpec((RA, H), lambda i: (i, 0)),
            pl.BlockSpec((2, RA, H), lambda i: (0, i, 0)),
        ],
        out_shape=[
            jax.ShapeDtypeStruct((N, H), jnp.float32),
            jax.ShapeDtypeStruct((2, N, H), jnp.float32),
        ],
    )


def _ln(h, g, b):
    mu = jnp.mean(h, axis=-1, keepdims=True)
    var = jnp.mean((h - mu) ** 2, axis=-1, keepdims=True)
    return (h - mu) / jnp.sqrt(var + 1e-5) * g + b


@functools.cache
def _stage_b(outer_ln, make_table):
    def body(*refs):
        it = iter(refs)
        p_ref, h_ref = next(it), next(it)
        w1, b1, g1, be1 = (next(it) for _ in range(4))
        w2, b2 = next(it), next(it)
        ng = nb = tn = None
        if outer_ln:
            ng, nb = next(it), next(it)
        if make_table:
            tn = next(it)
        hout_ref = next(it)
        tab_ref = next(it) if make_table else None

        p = p_ref[...]
        den = p[0]
        num = p[1]
        h = h_ref[...]
        out = num / (den + 1e-30) + h
        hm = jnp.dot(out, w1[...], preferred_element_type=jnp.float32)
        hm = jnp.maximum(_ln(hm + b1[...], g1[...], be1[...]), 0.0)
        h2 = jnp.dot(hm, w2[...], preferred_element_type=jnp.float32) + b2[...]
        if outer_ln:
            r = jnp.maximum(_ln(h2, ng[...], nb[...]), 0.0)
            hnew = h + r
        else:
            hnew = h2
        hout_ref[...] = hnew
        if make_table:
            tab_ref[...] = _table(hnew, tn[...])

    vec = lambda w: pl.BlockSpec((1, w), lambda i: (0, 0))
    in_specs = [
        pl.BlockSpec((NC, RA, H), lambda i: (0, i, 0)),
        pl.BlockSpec((RA, H), lambda i: (i, 0)),
        pl.BlockSpec((H, TH), lambda i: (0, 0)),
        vec(TH), vec(TH), vec(TH),
        pl.BlockSpec((TH, H), lambda i: (0, 0)),
        vec(H),
    ]
    if outer_ln:
        in_specs += [vec(H), vec(H)]
    if make_table:
        in_specs += [vec(H)]
    out_specs = [pl.BlockSpec((RA, H), lambda i: (i, 0))]
    out_shape = [jax.ShapeDtypeStruct((N, H), jnp.float32)]
    if make_table:
        out_specs.append(pl.BlockSpec((2, RA, H), lambda i: (0, i, 0)))
        out_shape.append(jax.ShapeDtypeStruct((2, N, H), jnp.float32))
    return pl.pallas_call(
        body, grid=(N // RA,), in_specs=in_specs,
        out_specs=out_specs, out_shape=out_shape)


def _stage_c1_body(h0, h1, h2, h3, wphi, bphi, wa, ba, wb, bb, wc, bc,
                   hp_ref, a_ref):
    xc = jnp.concatenate([h0[...], h1[...], h2[...], h3[...]], axis=1)
    hp = jnp.dot(xc, wphi[...], preferred_element_type=jnp.float32)
    hp = jnp.maximum(hp + bphi[...], 0.0)
    a = jnp.tanh(jnp.dot(hp, wa[...], preferred_element_type=jnp.float32)
                 + ba[...])
    b = jax.nn.sigmoid(jnp.dot(hp, wb[...], preferred_element_type=jnp.float32)
                       + bb[...])
    att = jnp.dot(a * b, wc[...], preferred_element_type=jnp.float32) + bc[...]
    hp_ref[...] = hp
    a_ref[...] = att


@functools.cache
def _stage_c1():
    hspec = lambda: pl.BlockSpec((RA, H), lambda i: (i, 0))
    wspec = lambda: pl.BlockSpec((D4, D4), lambda i: (0, 0))
    vspec = lambda: pl.BlockSpec((1, D4), lambda i: (0, 0))
    return pl.pallas_call(
        _stage_c1_body,
        grid=(N // RA,),
        in_specs=[
            hspec(), hspec(), hspec(), hspec(),
            wspec(), vspec(),
            wspec(), vspec(),
            wspec(), vspec(),
            pl.BlockSpec((D4, 1), lambda i: (0, 0)),
            pl.BlockSpec((1, 1), lambda i: (0, 0)),
        ],
        out_specs=[
            pl.BlockSpec((RA, D4), lambda i: (i, 0)),
            pl.BlockSpec((RA, 1), lambda i: (i, 0)),
        ],
        out_shape=[
            jax.ShapeDtypeStruct((N, D4), jnp.float32),
            jax.ShapeDtypeStruct((N, 1), jnp.float32),
        ],
    )


def _stage_c2_body(a_ref, bt_ref, hp_ref, wrho, brho, wr, br, risk_ref):
    att = a_ref[...]
    bt = bt_ref[...]
    hp = hp_ref[...]
    ms = []
    for g in range(G):
        mg = jnp.max(jnp.where(bt == g, att, -jnp.inf))
        ms.append(jnp.where(jnp.isfinite(mg), mg, 0.0))
    msel = jnp.zeros_like(att)
    for g in range(G):
        msel = jnp.where(bt == g, ms[g], msel)
    e = jnp.exp(att - msel)
    rows = []
    for g in range(G):
        w = jnp.where(bt == g, e, 0.0)
        sg = jnp.sum(w)
        pg = jnp.sum(hp * w, axis=0, keepdims=True)
        rows.append(pg / (sg + 1e-16))
    pooled = jnp.concatenate(rows, axis=0)
    hh = jnp.dot(pooled, wrho[...], preferred_element_type=jnp.float32)
    hh = jnp.maximum(hh + brho[...], 0.0)
    risk_ref[...] = (jnp.dot(hh, wr[...], preferred_element_type=jnp.float32)
                     + br[...])


@functools.cache
def _stage_c2():
    return pl.pallas_call(
        _stage_c2_body,
        grid=(1,),
        in_specs=[
            pl.BlockSpec((N, 1), lambda i: (0, 0)),
            pl.BlockSpec((N, 1), lambda i: (0, 0)),
            pl.BlockSpec((N, D4), lambda i: (0, 0)),
            pl.BlockSpec((D4, D4), lambda i: (0, 0)),
            pl.BlockSpec((1, D4), lambda i: (0, 0)),
            pl.BlockSpec((D4, 1), lambda i: (0, 0)),
            pl.BlockSpec((1, 1), lambda i: (0, 0)),
        ],
        out_specs=pl.BlockSpec((G, 1), lambda i: (0, 0)),
        out_shape=jax.ShapeDtypeStruct((G, 1), jnp.float32),
    )


def kernel(x, edge_index, batch, params):
    p = params
    src = edge_index[0].reshape(NS, NBLK, BLK)
    dst = edge_index[1].reshape(NS, NBLK, BLK)
    zeros = jnp.zeros((CH, H), jnp.float32)
    tvec = [jnp.broadcast_to(p["t%d" % l].reshape(1, 1), (1, H))
            for l in range(3)]

    h, tab = _stage_a()(x, p["W_fc"], p["b_fc"].reshape(1, H), tvec[0])
    hs = [h]
    for l in range(3):
        part = _edge_agg_kernel()(tab, src, dst, zeros)
        make_table = l < 2
        args = [part, h,
                p["W1_%d" % l], p["b1_%d" % l].reshape(1, TH),
                p["g1_%d" % l].reshape(1, TH), p["be1_%d" % l].reshape(1, TH),
                p["W2_%d" % l], p["b2_%d" % l].reshape(1, H)]
        if l > 0:
            args += [p["ng%d" % l].reshape(1, H), p["nb%d" % l].reshape(1, H)]
        if make_table:
            args += [tvec[l + 1]]
        outs = _stage_b(l > 0, make_table)(*args)
        if make_table:
            h, tab = outs
        else:
            h, = outs
        hs.append(h)

    hp, att = _stage_c1()(hs[0], hs[1], hs[2], hs[3],
                          p["W_phi"], p["b_phi"].reshape(1, D4),
                          p["Wa"], p["ba"].reshape(1, D4),
                          p["Wb"], p["bb"].reshape(1, D4),
                          p["Wc"], p["bc"].reshape(1, 1))
    risk = _stage_c2()(att, batch.reshape(N, 1), hp,
                       p["Wrho"], p["brho"].reshape(1, D4),
                       p["Wr"], p["br"].reshape(1, 1))
    return risk.reshape(G), att.reshape(1, N)


# speedup vs baseline: 1.1361x; 1.1361x over previous
import functools

import jax
import jax.numpy as jnp
from jax import lax
from jax.experimental import pallas as pl
from jax.experimental.pallas import tpu as pltpu
from jax.experimental.pallas import tpu_sc as plsc

N = 10000
E = 640000
H = 64
G = 4
D4 = 4 * H
TH = 2 * H

NC = 2
NS = 16
EPS = E // NS
BLK = 80
NBLK = EPS // BLK
CH = 632
CH_LAST = N - (NS - 1) * CH

RA = 1000
F_IN = 1536


@functools.cache
def _edge_agg_kernel():
    mesh = plsc.VectorSubcoreMesh(
        core_axis_name="c", subcore_axis_name="s",
        num_cores=NC, num_subcores=NS)

    @functools.partial(
        pl.kernel,
        out_type=jax.ShapeDtypeStruct((NC, N, H), jnp.float32),
        mesh=mesh,
        scratch_types=[
            pltpu.VMEM((128, BLK), jnp.int32),
            pltpu.VMEM((128, BLK), jnp.int32),
            pltpu.VMEM((BLK, H), jnp.float32),
            pltpu.VMEM((BLK, H), jnp.float32),
            pltpu.VMEM((BLK, H), jnp.float32),
            pltpu.VMEM((BLK, H), jnp.float32),
            pltpu.VMEM_SHARED((N, H), jnp.float32),
            pltpu.SemaphoreType.DMA,
            pltpu.SemaphoreType.DMA,
            pltpu.SemaphoreType.DMA,
            pltpu.SemaphoreType.DMA,
            pltpu.SemaphoreType.DMA,
            pltpu.SemaphoreType.DMA,
            pltpu.SemaphoreType.DMA,
            pltpu.SemaphoreType.DMA,
        ],
        compiler_params=pltpu.CompilerParams(use_tc_tiling_on_sc=False),
    )
    def edge_agg(tabs, src, dst, zeros, out, sidx, didx,
                 rows0, rows1, rows2, rows3, acc,
                 sg0, sg1, sg2, sg3, ss0, ss1, ss2, ss3):
        c = lax.axis_index("c")
        s = lax.axis_index("s")
        row0 = pl.multiple_of(s * CH, 8)
        @pl.when(s < NS - 1)
        def _zero_main():
            pltpu.sync_copy(zeros, acc.at[pl.ds(row0, CH)])

        @pl.when(s == NS - 1)
        def _zero_last():
            pltpu.sync_copy(zeros.at[pl.ds(0, CH_LAST)],
                            acc.at[pl.ds((NS - 1) * CH, CH_LAST)])

        plsc.subcore_barrier()

        rows = (rows0, rows1, rows2, rows3)
        sg = (sg0, sg1, sg2, sg3)
        ss = (ss0, ss1, ss2, ss3)

        def run(table):
            for off, nbh in ((0, 124), (124, 124), (248, 124), (372, 128)):
                pltpu.sync_copy(src.at[s, pl.ds(off, nbh)],
                                sidx.at[pl.ds(0, nbh)])
                pltpu.sync_copy(dst.at[s, pl.ds(off, nbh)],
                                didx.at[pl.ds(0, nbh)])
                for j in range(3):
                    pltpu.async_copy(table.at[sidx.at[j]], rows[j], sg[j])

                def body(k, carry):
                    for b in range(4):
                        i = 4 * k + b
                        pltpu.make_async_copy(table.at[sidx.at[i]], rows[b],
                                              sg[b]).wait()
                        pltpu.async_copy(rows[b], acc.at[didx.at[i]], ss[b],
                                         add=True)
                        bp = (b + 3) % 4

                        @pl.when(i >= 1)
                        def _drain_prev():
                            pltpu.make_async_copy(rows[bp],
                                                  acc.at[didx.at[i - 1]],
                                                  ss[bp]).wait()

                        @pl.when(i + 3 < nbh)
                        def _prefetch():
                            pltpu.async_copy(table.at[sidx.at[i + 3]],
                                             rows[bp], sg[bp])
                    return carry

                lax.fori_loop(0, nbh // 4, body, 0)
                last = (nbh - 1) % 4
                pltpu.make_async_copy(rows[last], acc.at[didx.at[nbh - 1]],
                                      ss[last]).wait()

        run(tabs.at[c])
        plsc.subcore_barrier()

        @pl.when(s < NS - 1)
        def _write_main():
            pltpu.sync_copy(acc.at[pl.ds(row0, CH)],
                            out.at[c, pl.ds(row0, CH)])

        @pl.when(s == NS - 1)
        def _write_last():
            pltpu.sync_copy(acc.at[pl.ds((NS - 1) * CH, CH_LAST)],
                            out.at[c, pl.ds((NS - 1) * CH, CH_LAST)])

    return edge_agg


def _table(h, tvec):
    y = jnp.maximum(h, 0.0) + 1e-7
    u = jnp.exp(tvec * y)
    return jnp.stack([u, y * u], axis=0)


def _stage_a_body(x_ref, w_ref, b_ref, t_ref, h_ref, tab_ref):
    h = jnp.dot(x_ref[...], w_ref[...], preferred_element_type=jnp.float32)
    h = jnp.maximum(h + b_ref[...], 0.0)
    h_ref[...] = h
    tab_ref[...] = _table(h, t_ref[...])


@functools.cache
def _stage_a():
    return pl.pallas_call(
        _stage_a_body,
        grid=(N // RA,),
        in_specs=[
            pl.BlockSpec((RA, F_IN), lambda i: (i, 0)),
            pl.BlockSpec((F_IN, H), lambda i: (0, 0)),
            pl.BlockSpec((1, H), lambda i: (0, 0)),
            pl.BlockSpec((1, H), lambda i: (0, 0)),
        ],
        out_specs=[
            pl.BlockSpec((RA, H), lambda i: (i, 0)),
            pl.BlockSpec((2, RA, H), lambda i: (0, i, 0)),
        ],
        out_shape=[
            jax.ShapeDtypeStruct((N, H), jnp.float32),
            jax.ShapeDtypeStruct((2, N, H), jnp.float32),
        ],
    )


def _ln(h, g, b):
    mu = jnp.mean(h, axis=-1, keepdims=True)
    var = jnp.mean((h - mu) ** 2, axis=-1, keepdims=True)
    return (h - mu) / jnp.sqrt(var + 1e-5) * g + b


@functools.cache
def _stage_b(outer_ln, make_table):
    def body(*refs):
        it = iter(refs)
        p_ref, h_ref = next(it), next(it)
        w1, b1, g1, be1 = (next(it) for _ in range(4))
        w2, b2 = next(it), next(it)
        ng = nb = tn = None
        if outer_ln:
            ng, nb = next(it), next(it)
        if make_table:
            tn = next(it)
        hout_ref = next(it)
        tab_ref = next(it) if make_table else None

        p = p_ref[...]
        den = p[0]
        num = p[1]
        h = h_ref[...]
        out = num / (den + 1e-30) + h
        hm = jnp.dot(out, w1[...], preferred_element_type=jnp.float32)
        hm = jnp.maximum(_ln(hm + b1[...], g1[...], be1[...]), 0.0)
        h2 = jnp.dot(hm, w2[...], preferred_element_type=jnp.float32) + b2[...]
        if outer_ln:
            r = jnp.maximum(_ln(h2, ng[...], nb[...]), 0.0)
            hnew = h + r
        else:
            hnew = h2
        hout_ref[...] = hnew
        if make_table:
            tab_ref[...] = _table(hnew, tn[...])

    vec = lambda w: pl.BlockSpec((1, w), lambda i: (0, 0))
    in_specs = [
        pl.BlockSpec((NC, RA, H), lambda i: (0, i, 0)),
        pl.BlockSpec((RA, H), lambda i: (i, 0)),
        pl.BlockSpec((H, TH), lambda i: (0, 0)),
        vec(TH), vec(TH), vec(TH),
        pl.BlockSpec((TH, H), lambda i: (0, 0)),
        vec(H),
    ]
    if outer_ln:
        in_specs += [vec(H), vec(H)]
    if make_table:
        in_specs += [vec(H)]
    out_specs = [pl.BlockSpec((RA, H), lambda i: (i, 0))]
    out_shape = [jax.ShapeDtypeStruct((N, H), jnp.float32)]
    if make_table:
        out_specs.append(pl.BlockSpec((2, RA, H), lambda i: (0, i, 0)))
        out_shape.append(jax.ShapeDtypeStruct((2, N, H), jnp.float32))
    return pl.pallas_call(
        body, grid=(N // RA,), in_specs=in_specs,
        out_specs=out_specs, out_shape=out_shape)


def _stage_c1_body(h0, h1, h2, h3, wphi, bphi, wa, ba, wb, bb, wc, bc,
                   hp_ref, a_ref):
    xc = jnp.concatenate([h0[...], h1[...], h2[...], h3[...]], axis=1)
    hp = jnp.dot(xc, wphi[...], preferred_element_type=jnp.float32)
    hp = jnp.maximum(hp + bphi[...], 0.0)
    a = jnp.tanh(jnp.dot(hp, wa[...], preferred_element_type=jnp.float32)
                 + ba[...])
    b = jax.nn.sigmoid(jnp.dot(hp, wb[...], preferred_element_type=jnp.float32)
                       + bb[...])
    att = jnp.dot(a * b, wc[...], preferred_element_type=jnp.float32) + bc[...]
    hp_ref[...] = hp
    a_ref[...] = att


@functools.cache
def _stage_c1():
    hspec = lambda: pl.BlockSpec((RA, H), lambda i: (i, 0))
    wspec = lambda: pl.BlockSpec((D4, D4), lambda i: (0, 0))
    vspec = lambda: pl.BlockSpec((1, D4), lambda i: (0, 0))
    return pl.pallas_call(
        _stage_c1_body,
        grid=(N // RA,),
        in_specs=[
            hspec(), hspec(), hspec(), hspec(),
            wspec(), vspec(),
            wspec(), vspec(),
            wspec(), vspec(),
            pl.BlockSpec((D4, 1), lambda i: (0, 0)),
            pl.BlockSpec((1, 1), lambda i: (0, 0)),
        ],
        out_specs=[
            pl.BlockSpec((RA, D4), lambda i: (i, 0)),
            pl.BlockSpec((RA, 1), lambda i: (i, 0)),
        ],
        out_shape=[
            jax.ShapeDtypeStruct((N, D4), jnp.float32),
            jax.ShapeDtypeStruct((N, 1), jnp.float32),
        ],
    )


def _stage_c2_body(a_ref, bt_ref, hp_ref, wrho, brho, wr, br, risk_ref):
    att = a_ref[...]
    bt = bt_ref[...]
    hp = hp_ref[...]
    ms = []
    for g in range(G):
        mg = jnp.max(jnp.where(bt == g, att, -jnp.inf))
        ms.append(jnp.where(jnp.isfinite(mg), mg, 0.0))
    msel = jnp.zeros_like(att)
    for g in range(G):
        msel = jnp.where(bt == g, ms[g], msel)
    e = jnp.exp(att - msel)
    rows = []
    for g in range(G):
        w = jnp.where(bt == g, e, 0.0)
        sg = jnp.sum(w)
        pg = jnp.sum(hp * w, axis=0, keepdims=True)
        rows.append(pg / (sg + 1e-16))
    pooled = jnp.concatenate(rows, axis=0)
    hh = jnp.dot(pooled, wrho[...], preferred_element_type=jnp.float32)
    hh = jnp.maximum(hh + brho[...], 0.0)
    risk_ref[...] = (jnp.dot(hh, wr[...], preferred_element_type=jnp.float32)
                     + br[...])


@functools.cache
def _stage_c2():
    return pl.pallas_call(
        _stage_c2_body,
        grid=(1,),
        in_specs=[
            pl.BlockSpec((N, 1), lambda i: (0, 0)),
            pl.BlockSpec((N, 1), lambda i: (0, 0)),
            pl.BlockSpec((N, D4), lambda i: (0, 0)),
            pl.BlockSpec((D4, D4), lambda i: (0, 0)),
            pl.BlockSpec((1, D4), lambda i: (0, 0)),
            pl.BlockSpec((D4, 1), lambda i: (0, 0)),
            pl.BlockSpec((1, 1), lambda i: (0, 0)),
        ],
        out_specs=pl.BlockSpec((G, 1), lambda i: (0, 0)),
        out_shape=jax.ShapeDtypeStruct((G, 1), jnp.float32),
    )


def kernel(x, edge_index, batch, params):
    p = params
    src = edge_index[0].reshape(NS, NBLK, BLK)
    dst = edge_index[1].reshape(NS, NBLK, BLK)
    zeros = jnp.zeros((CH, H), jnp.float32)
    tvec = [jnp.broadcast_to(p["t%d" % l].reshape(1, 1), (1, H))
            for l in range(3)]

    h, tab = _stage_a()(x, p["W_fc"], p["b_fc"].reshape(1, H), tvec[0])
    hs = [h]
    for l in range(3):
        part = _edge_agg_kernel()(tab, src, dst, zeros)
        make_table = l < 2
        args = [part, h,
                p["W1_%d" % l], p["b1_%d" % l].reshape(1, TH),
                p["g1_%d" % l].reshape(1, TH), p["be1_%d" % l].reshape(1, TH),
                p["W2_%d" % l], p["b2_%d" % l].reshape(1, H)]
        if l > 0:
            args += [p["ng%d" % l].reshape(1, H), p["nb%d" % l].reshape(1, H)]
        if make_table:
            args += [tvec[l + 1]]
        outs = _stage_b(l > 0, make_table)(*args)
        if make_table:
            h, tab = outs
        else:
            h, = outs
        hs.append(h)

    hp, att = _stage_c1()(hs[0], hs[1], hs[2], hs[3],
                          p["W_phi"], p["b_phi"].reshape(1, D4),
                          p["Wa"], p["ba"].reshape(1, D4),
                          p["Wb"], p["bb"].reshape(1, D4),
                          p["Wc"], p["bc"].reshape(1, 1))
    risk = _stage_c2()(att, batch.reshape(N, 1), hp,
                       p["Wrho"], p["brho"].reshape(1, D4),
                       p["Wr"], p["br"].reshape(1, 1))
    return risk.reshape(G), att.reshape(1, N)


# speedup vs baseline: 1.1548x; 1.0164x over previous
import functools

import jax
import jax.numpy as jnp
from jax import lax
from jax.experimental import pallas as pl
from jax.experimental.pallas import tpu as pltpu
from jax.experimental.pallas import tpu_sc as plsc

N = 10000
E = 640000
H = 64
G = 4
D4 = 4 * H
TH = 2 * H

NC = 2
NS = 16
EPS = E // NS
BLK = 80
NBLK = EPS // BLK
CH = 632
CH_LAST = N - (NS - 1) * CH

RA = 1000
F_IN = 1536


@functools.cache
def _edge_agg_kernel():
    mesh = plsc.VectorSubcoreMesh(
        core_axis_name="c", subcore_axis_name="s",
        num_cores=NC, num_subcores=NS)

    @functools.partial(
        pl.kernel,
        out_type=jax.ShapeDtypeStruct((NC, N, H), jnp.float32),
        mesh=mesh,
        scratch_types=[
            pltpu.VMEM((252, BLK), jnp.int32),
            pltpu.VMEM((252, BLK), jnp.int32),
            pltpu.VMEM((BLK, H), jnp.float32),
            pltpu.VMEM((BLK, H), jnp.float32),
            pltpu.VMEM((BLK, H), jnp.float32),
            pltpu.VMEM((BLK, H), jnp.float32),
            pltpu.VMEM_SHARED((N, H), jnp.float32),
            pltpu.SemaphoreType.DMA,
            pltpu.SemaphoreType.DMA,
            pltpu.SemaphoreType.DMA,
            pltpu.SemaphoreType.DMA,
            pltpu.SemaphoreType.DMA,
            pltpu.SemaphoreType.DMA,
            pltpu.SemaphoreType.DMA,
            pltpu.SemaphoreType.DMA,
        ],
        compiler_params=pltpu.CompilerParams(use_tc_tiling_on_sc=False),
    )
    def edge_agg(tabs, src, dst, zeros, out, sidx, didx,
                 rows0, rows1, rows2, rows3, acc,
                 sg0, sg1, sg2, sg3, ss0, ss1, ss2, ss3):
        c = lax.axis_index("c")
        s = lax.axis_index("s")
        row0 = pl.multiple_of(s * CH, 8)
        @pl.when(s < NS - 1)
        def _zero_main():
            pltpu.sync_copy(zeros, acc.at[pl.ds(row0, CH)])

        @pl.when(s == NS - 1)
        def _zero_last():
            pltpu.sync_copy(zeros.at[pl.ds(0, CH_LAST)],
                            acc.at[pl.ds((NS - 1) * CH, CH_LAST)])

        plsc.subcore_barrier()

        rows = (rows0, rows1, rows2, rows3)
        sg = (sg0, sg1, sg2, sg3)
        ss = (ss0, ss1, ss2, ss3)

        def run(table):
            for off, nbh in ((0, 248), (248, 252)):
                pltpu.sync_copy(src.at[s, pl.ds(off, nbh)],
                                sidx.at[pl.ds(0, nbh)])
                pltpu.sync_copy(dst.at[s, pl.ds(off, nbh)],
                                didx.at[pl.ds(0, nbh)])
                for j in range(3):
                    pltpu.async_copy(table.at[sidx.at[j]], rows[j], sg[j])

                def body(k, carry):
                    for b in range(4):
                        i = 4 * k + b
                        pltpu.make_async_copy(table.at[sidx.at[i]], rows[b],
                                              sg[b]).wait()
                        pltpu.async_copy(rows[b], acc.at[didx.at[i]], ss[b],
                                         add=True)
                        bp = (b + 3) % 4

                        @pl.when(i >= 1)
                        def _drain_prev():
                            pltpu.make_async_copy(rows[bp],
                                                  acc.at[didx.at[i - 1]],
                                                  ss[bp]).wait()

                        @pl.when(i + 3 < nbh)
                        def _prefetch():
                            pltpu.async_copy(table.at[sidx.at[i + 3]],
                                             rows[bp], sg[bp])
                    return carry

                lax.fori_loop(0, nbh // 4, body, 0)
                last = (nbh - 1) % 4
                pltpu.make_async_copy(rows[last], acc.at[didx.at[nbh - 1]],
                                      ss[last]).wait()

        run(tabs.at[c])
        plsc.subcore_barrier()

        @pl.when(s < NS - 1)
        def _write_main():
            pltpu.sync_copy(acc.at[pl.ds(row0, CH)],
                            out.at[c, pl.ds(row0, CH)])

        @pl.when(s == NS - 1)
        def _write_last():
            pltpu.sync_copy(acc.at[pl.ds((NS - 1) * CH, CH_LAST)],
                            out.at[c, pl.ds((NS - 1) * CH, CH_LAST)])

    return edge_agg


def _table(h, tvec):
    y = jnp.maximum(h, 0.0) + 1e-7
    u = jnp.exp(tvec * y)
    return jnp.stack([u, y * u], axis=0)


def _stage_a_body(x_ref, w_ref, b_ref, t_ref, h_ref, tab_ref):
    h = jnp.dot(x_ref[...], w_ref[...], preferred_element_type=jnp.float32)
    h = jnp.maximum(h + b_ref[...], 0.0)
    h_ref[...] = h
    tab_ref[...] = _table(h, t_ref[...])


@functools.cache
def _stage_a():
    return pl.pallas_call(
        _stage_a_body,
        grid=(N // RA,),
        in_specs=[
            pl.BlockSpec((RA, F_IN), lambda i: (i, 0)),
            pl.BlockSpec((F_IN, H), lambda i: (0, 0)),
            pl.BlockSpec((1, H), lambda i: (0, 0)),
            pl.BlockSpec((1, H), lambda i: (0, 0)),
        ],
        out_specs=[
            pl.BlockSpec((RA, H), lambda i: (i, 0)),
            pl.BlockSpec((2, RA, H), lambda i: (0, i, 0)),
        ],
        out_shape=[
            jax.ShapeDtypeStruct((N, H), jnp.float32),
            jax.ShapeDtypeStruct((2, N, H), jnp.float32),
        ],
    )


def _ln(h, g, b):
    mu = jnp.mean(h, axis=-1, keepdims=True)
    var = jnp.mean((h - mu) ** 2, axis=-1, keepdims=True)
    return (h - mu) / jnp.sqrt(var + 1e-5) * g + b


@functools.cache
def _stage_b(outer_ln, make_table):
    def body(*refs):
        it = iter(refs)
        p_ref, h_ref = next(it), next(it)
        w1, b1, g1, be1 = (next(it) for _ in range(4))
        w2, b2 = next(it), next(it)
        ng = nb = tn = None
        if outer_ln:
            ng, nb = next(it), next(it)
        if make_table:
            tn = next(it)
        hout_ref = next(it)
        tab_ref = next(it) if make_table else None

        p = p_ref[...]
        den = p[0]
        num = p[1]
        h = h_ref[...]
        out = num / (den + 1e-30) + h
        hm = jnp.dot(out, w1[...], preferred_element_type=jnp.float32)
        hm = jnp.maximum(_ln(hm + b1[...], g1[...], be1[...]), 0.0)
        h2 = jnp.dot(hm, w2[...], preferred_element_type=jnp.float32) + b2[...]
        if outer_ln:
            r = jnp.maximum(_ln(h2, ng[...], nb[...]), 0.0)
            hnew = h + r
        else:
            hnew = h2
        hout_ref[...] = hnew
        if make_table:
            tab_ref[...] = _table(hnew, tn[...])

    vec = lambda w: pl.BlockSpec((1, w), lambda i: (0, 0))
    in_specs = [
        pl.BlockSpec((NC, RA, H), lambda i: (0, i, 0)),
        pl.BlockSpec((RA, H), lambda i: (i, 0)),
        pl.BlockSpec((H, TH), lambda i: (0, 0)),
        vec(TH), vec(TH), vec(TH),
        pl.BlockSpec((TH, H), lambda i: (0, 0)),
        vec(H),
    ]
    if outer_ln:
        in_specs += [vec(H), vec(H)]
    if make_table:
        in_specs += [vec(H)]
    out_specs = [pl.BlockSpec((RA, H), lambda i: (i, 0))]
    out_shape = [jax.ShapeDtypeStruct((N, H), jnp.float32)]
    if make_table:
        out_specs.append(pl.BlockSpec((2, RA, H), lambda i: (0, i, 0)))
        out_shape.append(jax.ShapeDtypeStruct((2, N, H), jnp.float32))
    return pl.pallas_call(
        body, grid=(N // RA,), in_specs=in_specs,
        out_specs=out_specs, out_shape=out_shape)


def _stage_c1_body(h0, h1, h2, h3, wphi, bphi, wa, ba, wb, bb, wc, bc,
                   hp_ref, a_ref):
    xc = jnp.concatenate([h0[...], h1[...], h2[...], h3[...]], axis=1)
    hp = jnp.dot(xc, wphi[...], preferred_element_type=jnp.float32)
    hp = jnp.maximum(hp + bphi[...], 0.0)
    a = jnp.tanh(jnp.dot(hp, wa[...], preferred_element_type=jnp.float32)
                 + ba[...])
    b = jax.nn.sigmoid(jnp.dot(hp, wb[...], preferred_element_type=jnp.float32)
                       + bb[...])
    att = jnp.dot(a * b, wc[...], preferred_element_type=jnp.float32) + bc[...]
    hp_ref[...] = hp
    a_ref[...] = att


@functools.cache
def _stage_c1():
    hspec = lambda: pl.BlockSpec((RA, H), lambda i: (i, 0))
    wspec = lambda: pl.BlockSpec((D4, D4), lambda i: (0, 0))
    vspec = lambda: pl.BlockSpec((1, D4), lambda i: (0, 0))
    return pl.pallas_call(
        _stage_c1_body,
        grid=(N // RA,),
        in_specs=[
            hspec(), hspec(), hspec(), hspec(),
            wspec(), vspec(),
            wspec(), vspec(),
            wspec(), vspec(),
            pl.BlockSpec((D4, 1), lambda i: (0, 0)),
            pl.BlockSpec((1, 1), lambda i: (0, 0)),
        ],
        out_specs=[
            pl.BlockSpec((RA, D4), lambda i: (i, 0)),
            pl.BlockSpec((RA, 1), lambda i: (i, 0)),
        ],
        out_shape=[
            jax.ShapeDtypeStruct((N, D4), jnp.float32),
            jax.ShapeDtypeStruct((N, 1), jnp.float32),
        ],
    )


def _stage_c2_body(a_ref, bt_ref, hp_ref, wrho, brho, wr, br, risk_ref):
    att = a_ref[...]
    bt = bt_ref[...]
    hp = hp_ref[...]
    ms = []
    for g in range(G):
        mg = jnp.max(jnp.where(bt == g, att, -jnp.inf))
        ms.append(jnp.where(jnp.isfinite(mg), mg, 0.0))
    msel = jnp.zeros_like(att)
    for g in range(G):
        msel = jnp.where(bt == g, ms[g], msel)
    e = jnp.exp(att - msel)
    rows = []
    for g in range(G):
        w = jnp.where(bt == g, e, 0.0)
        sg = jnp.sum(w)
        pg = jnp.sum(hp * w, axis=0, keepdims=True)
        rows.append(pg / (sg + 1e-16))
    pooled = jnp.concatenate(rows, axis=0)
    hh = jnp.dot(pooled, wrho[...], preferred_element_type=jnp.float32)
    hh = jnp.maximum(hh + brho[...], 0.0)
    risk_ref[...] = (jnp.dot(hh, wr[...], preferred_element_type=jnp.float32)
                     + br[...])


@functools.cache
def _stage_c2():
    return pl.pallas_call(
        _stage_c2_body,
        grid=(1,),
        in_specs=[
            pl.BlockSpec((N, 1), lambda i: (0, 0)),
            pl.BlockSpec((N, 1), lambda i: (0, 0)),
            pl.BlockSpec((N, D4), lambda i: (0, 0)),
            pl.BlockSpec((D4, D4), lambda i: (0, 0)),
            pl.BlockSpec((1, D4), lambda i: (0, 0)),
            pl.BlockSpec((D4, 1), lambda i: (0, 0)),
            pl.BlockSpec((1, 1), lambda i: (0, 0)),
        ],
        out_specs=pl.BlockSpec((G, 1), lambda i: (0, 0)),
        out_shape=jax.ShapeDtypeStruct((G, 1), jnp.float32),
    )


def kernel(x, edge_index, batch, params):
    p = params
    src = edge_index[0].reshape(NS, NBLK, BLK)
    dst = edge_index[1].reshape(NS, NBLK, BLK)
    zeros = jnp.zeros((CH, H), jnp.float32)
    tvec = [jnp.broadcast_to(p["t%d" % l].reshape(1, 1), (1, H))
            for l in range(3)]

    h, tab = _stage_a()(x, p["W_fc"], p["b_fc"].reshape(1, H), tvec[0])
    hs = [h]
    for l in range(3):
        part = _edge_agg_kernel()(tab, src, dst, zeros)
        make_table = l < 2
        args = [part, h,
                p["W1_%d" % l], p["b1_%d" % l].reshape(1, TH),
                p["g1_%d" % l].reshape(1, TH), p["be1_%d" % l].reshape(1, TH),
                p["W2_%d" % l], p["b2_%d" % l].reshape(1, H)]
        if l > 0:
            args += [p["ng%d" % l].reshape(1, H), p["nb%d" % l].reshape(1, H)]
        if make_table:
            args += [tvec[l + 1]]
        outs = _stage_b(l > 0, make_table)(*args)
        if make_table:
            h, tab = outs
        else:
            h, = outs
        hs.append(h)

    hp, att = _stage_c1()(hs[0], hs[1], hs[2], hs[3],
                          p["W_phi"], p["b_phi"].reshape(1, D4),
                          p["Wa"], p["ba"].reshape(1, D4),
                          p["Wb"], p["bb"].reshape(1, D4),
                          p["Wc"], p["bc"].reshape(1, 1))
    risk = _stage_c2()(att, batch.reshape(N, 1), hp,
                       p["Wrho"], p["brho"].reshape(1, D4),
                       p["Wr"], p["br"].reshape(1, 1))
    return risk.reshape(G), att.reshape(1, N)
